# Initial kernel scaffold; baseline (speedup 1.0000x reference)
import jax
import jax.numpy as jnp
from jax.experimental import pallas as pl


def kernel(snp, snp_ids, node_graph_ids, filters, W1, b1, gamma1, beta1, W2, b2, gamma2, beta2, W3, b3):
    raise NotImplementedError("write your pallas kernel here")


# speedup vs baseline: 116.0214x; 116.0214x over previous
import functools

import jax
import jax.numpy as jnp
from jax import lax
from jax.experimental import pallas as pl
from jax.experimental.pallas import tpu as pltpu
from jax.experimental.pallas import tpu_sc as plsc

B = 32
N_SNPS = 50000
N_GENES = 10000
N_NODES = 160000
N_FILTERS = 8

R_BLK = 2000
N_SNPS_PAD = 52000

NUM_WORKERS = 32
CHUNK = 128
NODES_PER_W = 5120
NODES_PAD = NUM_WORKERS * NODES_PER_W
N_CHUNKS = NODES_PER_W // CHUNK
N_GENES_PAD = 10240
GENES_PER_TILE = N_GENES_PAD // 16

KB = 1000
N_KSTEPS = N_GENES // KB
H1, H2 = 1024, 256
EPS = 1e-5


def _prep_body(snpT_ref, filtT_ref, out_ref):
    fmean = jnp.mean(filtT_ref[...], axis=1, keepdims=True)
    out_ref[...] = snpT_ref[...] * fmean


def _make_prep():
    return pl.pallas_call(
        _prep_body,
        grid=(N_SNPS_PAD // R_BLK,),
        in_specs=[
            pl.BlockSpec((R_BLK, B), lambda i: (i, 0)),
            pl.BlockSpec((R_BLK, N_FILTERS), lambda i: (i, 0)),
        ],
        out_specs=pl.BlockSpec((R_BLK, B), lambda i: (i, 0)),
        out_shape=jax.ShapeDtypeStruct((N_SNPS_PAD, B), jnp.float32),
    )


def _sc_body(table_hbm, ids_hbm, genes_hbm, zeros_hbm, out_hbm,
             ids_v, genes_v, rows_v, acc_sh, sem):
    c = lax.axis_index("c")
    s = lax.axis_index("s")
    wid = s * 2 + c

    z0 = s * GENES_PER_TILE
    pltpu.sync_copy(zeros_hbm.at[pl.ds(z0, GENES_PER_TILE)],
                    acc_sh.at[pl.ds(z0, GENES_PER_TILE)])

    pltpu.sync_copy(ids_hbm.at[wid], ids_v)
    pltpu.sync_copy(genes_hbm.at[wid], genes_v)
    plsc.subcore_barrier()

    def chunk_step(j, carry):
        pltpu.async_copy(table_hbm.at[ids_v.at[j]], rows_v, sem).wait()
        pltpu.sync_copy(rows_v, acc_sh.at[genes_v.at[j]], add=True)
        return carry

    lax.fori_loop(0, N_CHUNKS, chunk_step, 0)
    plsc.subcore_barrier()

    row0 = c * N_GENES_PAD + z0
    pltpu.sync_copy(acc_sh.at[pl.ds(z0, GENES_PER_TILE)],
                    out_hbm.at[pl.ds(row0, GENES_PER_TILE)])


def _make_sc():
    mesh = plsc.VectorSubcoreMesh(core_axis_name="c", subcore_axis_name="s")
    return functools.partial(
        pl.kernel,
        out_type=jax.ShapeDtypeStruct((2 * N_GENES_PAD, B), jnp.float32),
        mesh=mesh,
        compiler_params=pltpu.CompilerParams(use_tc_tiling_on_sc=False),
        scratch_types=[
            pltpu.VMEM((N_CHUNKS, CHUNK), jnp.int32),
            pltpu.VMEM((N_CHUNKS, CHUNK), jnp.int32),
            pltpu.VMEM((CHUNK, B), jnp.float32),
            pltpu.VMEM_SHARED((N_GENES_PAD, B), jnp.float32),
            pltpu.SemaphoreType.DMA,
        ],
    )(_sc_body)


def _mlp_body(parts_ref, w1_ref, b1_ref, g1_ref, be1_ref,
              w2_ref, b2_ref, g2_ref, be2_ref, w3r_ref, b3_ref,
              out_ref, acc_ref):
    k = pl.program_id(0)

    @pl.when(k == 0)
    def _init():
        acc_ref[...] = jnp.zeros_like(acc_ref)

    x = parts_ref[0] + parts_ref[1]
    acc_ref[...] += lax.dot_general(
        x, w1_ref[...], (((0,), (0,)), ((), ())),
        preferred_element_type=jnp.float32)

    @pl.when(k == N_KSTEPS - 1)
    def _finish():
        h1 = acc_ref[...] + b1_ref[...]
        m1 = jnp.mean(h1, axis=0, keepdims=True)
        v1 = jnp.mean((h1 - m1) ** 2, axis=0, keepdims=True)
        h1 = g1_ref[...] * (h1 - m1) * lax.rsqrt(v1 + EPS) + be1_ref[...]
        h1 = jnp.maximum(h1, 0.0)
        h2 = jnp.dot(h1, w2_ref[...], preferred_element_type=jnp.float32) + b2_ref[...]
        m2 = jnp.mean(h2, axis=0, keepdims=True)
        v2 = jnp.mean((h2 - m2) ** 2, axis=0, keepdims=True)
        h2 = g2_ref[...] * (h2 - m2) * lax.rsqrt(v2 + EPS) + be2_ref[...]
        h2 = jnp.maximum(h2, 0.0)
        p = jnp.sum(h2 * w3r_ref[...], axis=1, keepdims=True)
        out_ref[...] = p + b3_ref[...]


def _make_mlp():
    full = lambda i: (0, 0)
    return pl.pallas_call(
        _mlp_body,
        grid=(N_KSTEPS,),
        in_specs=[
            pl.BlockSpec((2, KB, B), lambda i: (0, i, 0)),
            pl.BlockSpec((KB, H1), lambda i: (i, 0)),
            pl.BlockSpec((1, H1), full),
            pl.BlockSpec((1, H1), full),
            pl.BlockSpec((1, H1), full),
            pl.BlockSpec((H1, H2), full),
            pl.BlockSpec((1, H2), full),
            pl.BlockSpec((1, H2), full),
            pl.BlockSpec((1, H2), full),
            pl.BlockSpec((1, H2), full),
            pl.BlockSpec((1, 128), full),
        ],
        out_specs=pl.BlockSpec((B, 128), full),
        out_shape=jax.ShapeDtypeStruct((B, 128), jnp.float32),
        scratch_shapes=[pltpu.VMEM((B, H1), jnp.float32)],
    )


def kernel(snp, snp_ids, node_graph_ids, filters, W1, b1, gamma1, beta1,
           W2, b2, gamma2, beta2, W3, b3):
    f32 = jnp.float32

    snpT = jnp.zeros((N_SNPS_PAD, B), f32).at[:N_SNPS].set(snp.T)
    filtT = jnp.zeros((N_SNPS_PAD, N_FILTERS), f32).at[:N_SNPS].set(filters.T)
    table = _make_prep()(snpT, filtT)

    pad = NODES_PAD - N_NODES
    ids_p = jnp.concatenate(
        [snp_ids, jnp.full((pad,), N_SNPS, jnp.int32)]).reshape(
            NUM_WORKERS, N_CHUNKS, CHUNK)
    genes_p = jnp.concatenate(
        [node_graph_ids, jnp.full((pad,), N_GENES - 1, jnp.int32)]).reshape(
            NUM_WORKERS, N_CHUNKS, CHUNK)
    zeros_init = jnp.zeros((N_GENES_PAD, B), f32)

    parts = _make_sc()(table, ids_p, genes_p, zeros_init)
    parts = parts.reshape(2, N_GENES_PAD, B)

    out = _make_mlp()(
        parts, W1,
        b1.reshape(1, H1), gamma1.reshape(1, H1), beta1.reshape(1, H1),
        W2, b2.reshape(1, H2), gamma2.reshape(1, H2), beta2.reshape(1, H2),
        W3.reshape(1, H2), jnp.broadcast_to(b3.reshape(1, 1), (1, 128)),
    )
    preds = out[:, :1]
    return (preds, filters)


# speedup vs baseline: 126.6296x; 1.0914x over previous
import functools

import jax
import jax.numpy as jnp
from jax import lax
from jax.experimental import pallas as pl
from jax.experimental.pallas import tpu as pltpu
from jax.experimental.pallas import tpu_sc as plsc

B = 32
N_SNPS = 50000
N_GENES = 10000
N_NODES = 160000
N_FILTERS = 8

R_BLK = 2000
N_SNPS_PAD = 52000

NUM_WORKERS = 32
CHUNK = 128
NODES_PER_W = 5120
NODES_PAD = NUM_WORKERS * NODES_PER_W
N_CHUNKS = NODES_PER_W // CHUNK
N_GENES_PAD = 10240
GENES_PER_TILE = N_GENES_PAD // 16

KB = 1000
N_KSTEPS = N_GENES // KB
H1, H2 = 1024, 256
EPS = 1e-5


def _prep_body(snpT_ref, filtT_ref, out_ref):
    fmean = jnp.mean(filtT_ref[...], axis=1, keepdims=True)
    out_ref[...] = snpT_ref[...] * fmean


def _make_prep():
    return pl.pallas_call(
        _prep_body,
        grid=(N_SNPS_PAD // R_BLK,),
        in_specs=[
            pl.BlockSpec((R_BLK, B), lambda i: (i, 0)),
            pl.BlockSpec((R_BLK, N_FILTERS), lambda i: (i, 0)),
        ],
        out_specs=pl.BlockSpec((R_BLK, B), lambda i: (i, 0)),
        out_shape=jax.ShapeDtypeStruct((N_SNPS_PAD, B), jnp.float32),
    )


K_GRP = 4
N_GRPS = N_CHUNKS // K_GRP


def _sc_body(table_hbm, ids_hbm, genes_hbm, zeros_hbm, out_hbm,
             ids_v, genes_v, rows_v, acc_sh, gsems, ssems):
    c = lax.axis_index("c")
    s = lax.axis_index("s")
    wid = s * 2 + c

    pltpu.sync_copy(ids_hbm.at[wid], ids_v)
    pltpu.sync_copy(genes_hbm.at[wid], genes_v)

    def g_desc(t, p, b):
        return pltpu.make_async_copy(
            table_hbm.at[ids_v.at[t * K_GRP + b]],
            rows_v.at[p * K_GRP + b], gsems.at[p])

    def s_start(t, p, b):
        pltpu.async_copy(rows_v.at[p * K_GRP + b],
                         acc_sh.at[genes_v.at[t * K_GRP + b]],
                         ssems.at[p], add=True)

    def s_wait(t, p, b):
        pltpu.make_async_copy(rows_v.at[p * K_GRP + b],
                              acc_sh.at[genes_v.at[t * K_GRP + b]],
                              ssems.at[p]).wait()

    for b in range(K_GRP):
        g_desc(0, 0, b).start()

    z0 = s * GENES_PER_TILE
    pltpu.sync_copy(zeros_hbm.at[pl.ds(z0, GENES_PER_TILE)],
                    acc_sh.at[pl.ds(z0, GENES_PER_TILE)])
    plsc.subcore_barrier()

    for t in range(N_GRPS):
        p = t % 2
        for b in range(K_GRP):
            g_desc(t, p, b).wait()
        for b in range(K_GRP):
            s_start(t, p, b)
        if t >= 1:
            for b in range(K_GRP):
                s_wait(t - 1, 1 - p, b)
        if t + 1 < N_GRPS:
            for b in range(K_GRP):
                g_desc(t + 1, 1 - p, b).start()
    for b in range(K_GRP):
        s_wait(N_GRPS - 1, (N_GRPS - 1) % 2, b)
    plsc.subcore_barrier()

    row0 = c * N_GENES_PAD + z0
    pltpu.sync_copy(acc_sh.at[pl.ds(z0, GENES_PER_TILE)],
                    out_hbm.at[pl.ds(row0, GENES_PER_TILE)])


def _make_sc():
    mesh = plsc.VectorSubcoreMesh(core_axis_name="c", subcore_axis_name="s")
    return functools.partial(
        pl.kernel,
        out_type=jax.ShapeDtypeStruct((2 * N_GENES_PAD, B), jnp.float32),
        mesh=mesh,
        compiler_params=pltpu.CompilerParams(use_tc_tiling_on_sc=False),
        scratch_types=[
            pltpu.VMEM((N_CHUNKS, CHUNK), jnp.int32),
            pltpu.VMEM((N_CHUNKS, CHUNK), jnp.int32),
            pltpu.VMEM((2 * K_GRP, CHUNK, B), jnp.float32),
            pltpu.VMEM_SHARED((N_GENES_PAD, B), jnp.float32),
            pltpu.SemaphoreType.DMA((2,)),
            pltpu.SemaphoreType.DMA((2,)),
        ],
    )(_sc_body)


def _mlp_body(parts_ref, w1_ref, b1_ref, g1_ref, be1_ref,
              w2_ref, b2_ref, g2_ref, be2_ref, w3r_ref, b3_ref,
              out_ref, acc_ref):
    k = pl.program_id(0)

    @pl.when(k == 0)
    def _init():
        acc_ref[...] = jnp.zeros_like(acc_ref)

    x = parts_ref[0] + parts_ref[1]
    acc_ref[...] += lax.dot_general(
        x, w1_ref[...], (((0,), (0,)), ((), ())),
        preferred_element_type=jnp.float32)

    @pl.when(k == N_KSTEPS - 1)
    def _finish():
        h1 = acc_ref[...] + b1_ref[...]
        m1 = jnp.mean(h1, axis=0, keepdims=True)
        v1 = jnp.mean((h1 - m1) ** 2, axis=0, keepdims=True)
        h1 = g1_ref[...] * (h1 - m1) * lax.rsqrt(v1 + EPS) + be1_ref[...]
        h1 = jnp.maximum(h1, 0.0)
        h2 = jnp.dot(h1, w2_ref[...], preferred_element_type=jnp.float32) + b2_ref[...]
        m2 = jnp.mean(h2, axis=0, keepdims=True)
        v2 = jnp.mean((h2 - m2) ** 2, axis=0, keepdims=True)
        h2 = g2_ref[...] * (h2 - m2) * lax.rsqrt(v2 + EPS) + be2_ref[...]
        h2 = jnp.maximum(h2, 0.0)
        p = jnp.sum(h2 * w3r_ref[...], axis=1, keepdims=True)
        out_ref[...] = p + b3_ref[...]


def _make_mlp():
    full = lambda i: (0, 0)
    return pl.pallas_call(
        _mlp_body,
        grid=(N_KSTEPS,),
        in_specs=[
            pl.BlockSpec((2, KB, B), lambda i: (0, i, 0)),
            pl.BlockSpec((KB, H1), lambda i: (i, 0)),
            pl.BlockSpec((1, H1), full),
            pl.BlockSpec((1, H1), full),
            pl.BlockSpec((1, H1), full),
            pl.BlockSpec((H1, H2), full),
            pl.BlockSpec((1, H2), full),
            pl.BlockSpec((1, H2), full),
            pl.BlockSpec((1, H2), full),
            pl.BlockSpec((1, H2), full),
            pl.BlockSpec((1, 128), full),
        ],
        out_specs=pl.BlockSpec((B, 128), full),
        out_shape=jax.ShapeDtypeStruct((B, 128), jnp.float32),
        scratch_shapes=[pltpu.VMEM((B, H1), jnp.float32)],
    )


def kernel(snp, snp_ids, node_graph_ids, filters, W1, b1, gamma1, beta1,
           W2, b2, gamma2, beta2, W3, b3):
    f32 = jnp.float32

    snpT = jnp.zeros((N_SNPS_PAD, B), f32).at[:N_SNPS].set(snp.T)
    filtT = jnp.zeros((N_SNPS_PAD, N_FILTERS), f32).at[:N_SNPS].set(filters.T)
    table = _make_prep()(snpT, filtT)

    pad = NODES_PAD - N_NODES
    ids_p = jnp.concatenate(
        [snp_ids, jnp.full((pad,), N_SNPS, jnp.int32)]).reshape(
            NUM_WORKERS, N_CHUNKS, CHUNK)
    genes_p = jnp.concatenate(
        [node_graph_ids, jnp.full((pad,), N_GENES - 1, jnp.int32)]).reshape(
            NUM_WORKERS, N_CHUNKS, CHUNK)
    zeros_init = jnp.zeros((N_GENES_PAD, B), f32)

    parts = _make_sc()(table, ids_p, genes_p, zeros_init)
    parts = parts.reshape(2, N_GENES_PAD, B)

    out = _make_mlp()(
        parts, W1,
        b1.reshape(1, H1), gamma1.reshape(1, H1), beta1.reshape(1, H1),
        W2, b2.reshape(1, H2), gamma2.reshape(1, H2), beta2.reshape(1, H2),
        W3.reshape(1, H2), jnp.broadcast_to(b3.reshape(1, 1), (1, 128)),
    )
    preds = out[:, :1]
    return (preds, filters)


# speedup vs baseline: 194.3176x; 1.5345x over previous
import functools

import jax
import jax.numpy as jnp
from jax import lax
from jax.experimental import pallas as pl
from jax.experimental.pallas import tpu as pltpu
from jax.experimental.pallas import tpu_sc as plsc

B = 32
N_SNPS = 50000
N_GENES = 10000
N_NODES = 160000
N_FILTERS = 8

R_BLK = 2000
N_SNPS_PAD = 52000

CHUNK = 128
TOT_CHUNKS = 1280
NODES_PAD = TOT_CHUNKS * CHUNK
K_GRP = 4
C0 = 60
C1 = 20
G0 = C0 // K_GRP
G1 = C1 // K_GRP
N_GENES_PAD = 10240
GENES_PER_TILE = N_GENES_PAD // 16

KB = 1000
N_KSTEPS = N_GENES // KB
H1, H2 = 1024, 256
EPS = 1e-5


def _prep_body(snp_ref, filt_ref, out_ref):
    fmean = jnp.mean(filt_ref[...], axis=0, keepdims=True)
    out_ref[pl.ds(0, N_SNPS), :] = jnp.transpose(snp_ref[...] * fmean)
    out_ref[pl.ds(N_SNPS, N_SNPS_PAD - N_SNPS), :] = jnp.zeros(
        (N_SNPS_PAD - N_SNPS, B), jnp.float32)


def _make_prep():
    return pl.pallas_call(
        _prep_body,
        out_shape=jax.ShapeDtypeStruct((N_SNPS_PAD, B), jnp.float32),
    )


def _sc_body(table_hbm, ids_hbm, genes_hbm, zeros_hbm, out_hbm,
             ids_v, genes_v, rows_v, acc_sh, gsems, ssems):
    c = lax.axis_index("c")
    s = lax.axis_index("s")
    is0 = c == 0
    ngroups = jnp.where(is0, G0, G1)

    @pl.when(is0)
    def _load0():
        pltpu.sync_copy(ids_hbm.at[pl.ds(s * C0, C0)], ids_v.at[pl.ds(0, C0)])
        pltpu.sync_copy(genes_hbm.at[pl.ds(s * C0, C0)],
                        genes_v.at[pl.ds(0, C0)])

    @pl.when(~is0)
    def _load1():
        base = 16 * C0 + s * C1
        pltpu.sync_copy(ids_hbm.at[pl.ds(base, C1)], ids_v.at[pl.ds(0, C1)])
        pltpu.sync_copy(genes_hbm.at[pl.ds(base, C1)],
                        genes_v.at[pl.ds(0, C1)])

    def g_desc(t, p, b):
        return pltpu.make_async_copy(
            table_hbm.at[ids_v.at[t * K_GRP + b]],
            rows_v.at[p * K_GRP + b], gsems.at[p])

    def s_start(t, p, b):
        pltpu.async_copy(rows_v.at[p * K_GRP + b],
                         acc_sh.at[genes_v.at[t * K_GRP + b]],
                         ssems.at[p], add=True)

    def s_wait(t, p, b):
        pltpu.make_async_copy(rows_v.at[p * K_GRP + b],
                              acc_sh.at[genes_v.at[t * K_GRP + b]],
                              ssems.at[p]).wait()

    for b in range(K_GRP):
        g_desc(0, 0, b).start()

    z0 = s * GENES_PER_TILE
    pltpu.sync_copy(zeros_hbm.at[pl.ds(z0, GENES_PER_TILE)],
                    acc_sh.at[pl.ds(z0, GENES_PER_TILE)])
    plsc.subcore_barrier()

    def group_step(t, carry):
        p = t % 2
        for b in range(K_GRP):
            g_desc(t, p, b).wait()
        for b in range(K_GRP):
            s_start(t, p, b)

        @pl.when(t >= 1)
        def _drain_prev():
            for b in range(K_GRP):
                s_wait(t - 1, 1 - p, b)

        @pl.when(t + 1 < ngroups)
        def _fire_next():
            for b in range(K_GRP):
                g_desc(t + 1, 1 - p, b).start()

        return carry

    lax.fori_loop(0, ngroups, group_step, 0)
    for b in range(K_GRP):
        s_wait(ngroups - 1, (ngroups - 1) % 2, b)
    plsc.subcore_barrier()

    row0 = c * N_GENES_PAD + z0
    pltpu.sync_copy(acc_sh.at[pl.ds(z0, GENES_PER_TILE)],
                    out_hbm.at[pl.ds(row0, GENES_PER_TILE)])


def _make_sc():
    mesh = plsc.VectorSubcoreMesh(core_axis_name="c", subcore_axis_name="s")
    return functools.partial(
        pl.kernel,
        out_type=jax.ShapeDtypeStruct((2 * N_GENES_PAD, B), jnp.float32),
        mesh=mesh,
        compiler_params=pltpu.CompilerParams(use_tc_tiling_on_sc=False),
        scratch_types=[
            pltpu.VMEM((C0, CHUNK), jnp.int32),
            pltpu.VMEM((C0, CHUNK), jnp.int32),
            pltpu.VMEM((2 * K_GRP, CHUNK, B), jnp.float32),
            pltpu.VMEM_SHARED((N_GENES_PAD, B), jnp.float32),
            pltpu.SemaphoreType.DMA((2,)),
            pltpu.SemaphoreType.DMA((2,)),
        ],
    )(_sc_body)


def _mlp_body(parts_ref, w1_ref, b1_ref, g1_ref, be1_ref,
              w2_ref, b2_ref, g2_ref, be2_ref, w3r_ref, b3_ref,
              out_ref, acc_ref):
    k = pl.program_id(0)

    @pl.when(k == 0)
    def _init():
        acc_ref[...] = jnp.zeros_like(acc_ref)

    x = parts_ref[0] + parts_ref[1]
    acc_ref[...] += lax.dot_general(
        x, w1_ref[...], (((0,), (0,)), ((), ())),
        preferred_element_type=jnp.float32)

    @pl.when(k == N_KSTEPS - 1)
    def _finish():
        h1 = acc_ref[...] + b1_ref[...]
        m1 = jnp.mean(h1, axis=0, keepdims=True)
        v1 = jnp.mean((h1 - m1) ** 2, axis=0, keepdims=True)
        h1 = g1_ref[...] * (h1 - m1) * lax.rsqrt(v1 + EPS) + be1_ref[...]
        h1 = jnp.maximum(h1, 0.0)
        h2 = jnp.dot(h1, w2_ref[...], preferred_element_type=jnp.float32) + b2_ref[...]
        m2 = jnp.mean(h2, axis=0, keepdims=True)
        v2 = jnp.mean((h2 - m2) ** 2, axis=0, keepdims=True)
        h2 = g2_ref[...] * (h2 - m2) * lax.rsqrt(v2 + EPS) + be2_ref[...]
        h2 = jnp.maximum(h2, 0.0)
        p = jnp.sum(h2 * w3r_ref[...], axis=1, keepdims=True)
        out_ref[...] = p + b3_ref[...]


def _make_mlp():
    full = lambda i: (0, 0)
    return pl.pallas_call(
        _mlp_body,
        grid=(N_KSTEPS,),
        in_specs=[
            pl.BlockSpec((2, KB, B), lambda i: (0, i, 0)),
            pl.BlockSpec((KB, H1), lambda i: (i, 0)),
            pl.BlockSpec((1, H1), full),
            pl.BlockSpec((1, H1), full),
            pl.BlockSpec((1, H1), full),
            pl.BlockSpec((H1, H2), full),
            pl.BlockSpec((1, H2), full),
            pl.BlockSpec((1, H2), full),
            pl.BlockSpec((1, H2), full),
            pl.BlockSpec((1, H2), full),
            pl.BlockSpec((1, 128), full),
        ],
        out_specs=pl.BlockSpec((B, 128), full),
        out_shape=jax.ShapeDtypeStruct((B, 128), jnp.float32),
        scratch_shapes=[pltpu.VMEM((B, H1), jnp.float32)],
    )


def kernel(snp, snp_ids, node_graph_ids, filters, W1, b1, gamma1, beta1,
           W2, b2, gamma2, beta2, W3, b3):
    f32 = jnp.float32

    table = _make_prep()(snp, filters)

    pad = NODES_PAD - N_NODES
    ids_p = jnp.concatenate(
        [snp_ids, jnp.full((pad,), N_SNPS, jnp.int32)]).reshape(
            TOT_CHUNKS, CHUNK)
    genes_p = jnp.concatenate(
        [node_graph_ids, jnp.full((pad,), N_GENES - 1, jnp.int32)]).reshape(
            TOT_CHUNKS, CHUNK)
    zeros_init = jnp.zeros((N_GENES_PAD, B), f32)

    parts = _make_sc()(table, ids_p, genes_p, zeros_init)
    parts = parts.reshape(2, N_GENES_PAD, B)

    out = _make_mlp()(
        parts, W1,
        b1.reshape(1, H1), gamma1.reshape(1, H1), beta1.reshape(1, H1),
        W2, b2.reshape(1, H2), gamma2.reshape(1, H2), beta2.reshape(1, H2),
        W3.reshape(1, H2), jnp.broadcast_to(b3.reshape(1, 1), (1, 128)),
    )
    preds = out[:, :1]
    return (preds, filters)


# speedup vs baseline: 195.6513x; 1.0069x over previous
import functools

import jax
import jax.numpy as jnp
from jax import lax
from jax.experimental import pallas as pl
from jax.experimental.pallas import tpu as pltpu
from jax.experimental.pallas import tpu_sc as plsc

B = 32
N_SNPS = 50000
N_GENES = 10000
N_NODES = 160000
N_FILTERS = 8

R_BLK = 2000
N_SNPS_PAD = 52000

CHUNK = 128
TOT_CHUNKS = 1280
NODES_PAD = TOT_CHUNKS * CHUNK
K_GRP = 4
C0 = 60
C1 = 20
G0 = C0 // K_GRP
G1 = C1 // K_GRP
N_GENES_PAD = 10240
GENES_PER_TILE = N_GENES_PAD // 16

KB = 1000
N_KSTEPS = N_GENES // KB
H1, H2 = 1024, 256
EPS = 1e-5


def _prep_body(snp_ref, filt_ref, out_ref):
    fmean = jnp.mean(filt_ref[...], axis=0, keepdims=True)
    out_ref[pl.ds(0, N_SNPS), :] = jnp.transpose(snp_ref[...] * fmean)
    out_ref[pl.ds(N_SNPS, N_SNPS_PAD - N_SNPS), :] = jnp.zeros(
        (N_SNPS_PAD - N_SNPS, B), jnp.float32)


def _make_prep():
    return pl.pallas_call(
        _prep_body,
        out_shape=jax.ShapeDtypeStruct((N_SNPS_PAD, B), jnp.float32),
    )


def _sc_body(table_hbm, ids_hbm, genes_hbm, zeros_hbm, out_hbm,
             ids_v, genes_v, rows_v, acc_sh, gsems, ssems, isems):
    c = lax.axis_index("c")
    s = lax.axis_index("s")
    is0 = c == 0
    ngroups = jnp.where(is0, G0, G1)

    z0 = s * GENES_PER_TILE
    zero_cp = pltpu.make_async_copy(
        zeros_hbm.at[pl.ds(z0, GENES_PER_TILE)],
        acc_sh.at[pl.ds(z0, GENES_PER_TILE)], isems.at[2])
    zero_cp.start()

    @pl.when(is0)
    def _load0():
        pltpu.async_copy(ids_hbm.at[pl.ds(s * C0, C0)],
                         ids_v.at[pl.ds(0, C0)], isems.at[0])
        pltpu.async_copy(genes_hbm.at[pl.ds(s * C0, C0)],
                         genes_v.at[pl.ds(0, C0)], isems.at[1])
        pltpu.make_async_copy(ids_hbm.at[pl.ds(s * C0, C0)],
                              ids_v.at[pl.ds(0, C0)], isems.at[0]).wait()
        pltpu.make_async_copy(genes_hbm.at[pl.ds(s * C0, C0)],
                              genes_v.at[pl.ds(0, C0)], isems.at[1]).wait()

    @pl.when(~is0)
    def _load1():
        base = 16 * C0 + s * C1
        pltpu.async_copy(ids_hbm.at[pl.ds(base, C1)],
                         ids_v.at[pl.ds(0, C1)], isems.at[0])
        pltpu.async_copy(genes_hbm.at[pl.ds(base, C1)],
                         genes_v.at[pl.ds(0, C1)], isems.at[1])
        pltpu.make_async_copy(ids_hbm.at[pl.ds(base, C1)],
                              ids_v.at[pl.ds(0, C1)], isems.at[0]).wait()
        pltpu.make_async_copy(genes_hbm.at[pl.ds(base, C1)],
                              genes_v.at[pl.ds(0, C1)], isems.at[1]).wait()

    def g_desc(t, p, b):
        return pltpu.make_async_copy(
            table_hbm.at[ids_v.at[t * K_GRP + b]],
            rows_v.at[p * K_GRP + b], gsems.at[p])

    def s_start(t, p, b):
        pltpu.async_copy(rows_v.at[p * K_GRP + b],
                         acc_sh.at[genes_v.at[t * K_GRP + b]],
                         ssems.at[p], add=True)

    def s_wait(t, p, b):
        pltpu.make_async_copy(rows_v.at[p * K_GRP + b],
                              acc_sh.at[genes_v.at[t * K_GRP + b]],
                              ssems.at[p]).wait()

    for b in range(K_GRP):
        g_desc(0, 0, b).start()

    zero_cp.wait()
    plsc.subcore_barrier()

    def group_step(t, carry):
        p = t % 2
        for b in range(K_GRP):
            g_desc(t, p, b).wait()
        for b in range(K_GRP):
            s_start(t, p, b)

        @pl.when(t >= 1)
        def _drain_prev():
            for b in range(K_GRP):
                s_wait(t - 1, 1 - p, b)

        @pl.when(t + 1 < ngroups)
        def _fire_next():
            for b in range(K_GRP):
                g_desc(t + 1, 1 - p, b).start()

        return carry

    lax.fori_loop(0, ngroups, group_step, 0)
    for b in range(K_GRP):
        s_wait(ngroups - 1, (ngroups - 1) % 2, b)
    plsc.subcore_barrier()

    row0 = c * N_GENES_PAD + z0
    pltpu.sync_copy(acc_sh.at[pl.ds(z0, GENES_PER_TILE)],
                    out_hbm.at[pl.ds(row0, GENES_PER_TILE)])


def _make_sc():
    mesh = plsc.VectorSubcoreMesh(core_axis_name="c", subcore_axis_name="s")
    return functools.partial(
        pl.kernel,
        out_type=jax.ShapeDtypeStruct((2 * N_GENES_PAD, B), jnp.float32),
        mesh=mesh,
        compiler_params=pltpu.CompilerParams(use_tc_tiling_on_sc=False),
        scratch_types=[
            pltpu.VMEM((C0, CHUNK), jnp.int32),
            pltpu.VMEM((C0, CHUNK), jnp.int32),
            pltpu.VMEM((2 * K_GRP, CHUNK, B), jnp.float32),
            pltpu.VMEM_SHARED((N_GENES_PAD, B), jnp.float32),
            pltpu.SemaphoreType.DMA((2,)),
            pltpu.SemaphoreType.DMA((2,)),
            pltpu.SemaphoreType.DMA((3,)),
        ],
    )(_sc_body)


def _mlp_body(parts_ref, w1_ref, b1_ref, g1_ref, be1_ref,
              w2_ref, b2_ref, g2_ref, be2_ref, w3r_ref, b3_ref,
              out_ref, acc_ref):
    k = pl.program_id(0)

    @pl.when(k == 0)
    def _init():
        acc_ref[...] = jnp.zeros_like(acc_ref)

    x = parts_ref[0] + parts_ref[1]
    acc_ref[...] += lax.dot_general(
        x, w1_ref[...], (((0,), (0,)), ((), ())),
        preferred_element_type=jnp.float32)

    @pl.when(k == N_KSTEPS - 1)
    def _finish():
        h1 = acc_ref[...] + b1_ref[...]
        m1 = jnp.mean(h1, axis=0, keepdims=True)
        v1 = jnp.mean((h1 - m1) ** 2, axis=0, keepdims=True)
        h1 = g1_ref[...] * (h1 - m1) * lax.rsqrt(v1 + EPS) + be1_ref[...]
        h1 = jnp.maximum(h1, 0.0)
        h2 = jnp.dot(h1, w2_ref[...], preferred_element_type=jnp.float32) + b2_ref[...]
        m2 = jnp.mean(h2, axis=0, keepdims=True)
        v2 = jnp.mean((h2 - m2) ** 2, axis=0, keepdims=True)
        h2 = g2_ref[...] * (h2 - m2) * lax.rsqrt(v2 + EPS) + be2_ref[...]
        h2 = jnp.maximum(h2, 0.0)
        p = jnp.sum(h2 * w3r_ref[...], axis=1, keepdims=True)
        out_ref[...] = p + b3_ref[...]


def _make_mlp():
    full = lambda i: (0, 0)
    return pl.pallas_call(
        _mlp_body,
        grid=(N_KSTEPS,),
        in_specs=[
            pl.BlockSpec((2, KB, B), lambda i: (0, i, 0)),
            pl.BlockSpec((KB, H1), lambda i: (i, 0)),
            pl.BlockSpec((1, H1), full),
            pl.BlockSpec((1, H1), full),
            pl.BlockSpec((1, H1), full),
            pl.BlockSpec((H1, H2), full),
            pl.BlockSpec((1, H2), full),
            pl.BlockSpec((1, H2), full),
            pl.BlockSpec((1, H2), full),
            pl.BlockSpec((1, H2), full),
            pl.BlockSpec((1, 128), full),
        ],
        out_specs=pl.BlockSpec((B, 128), full),
        out_shape=jax.ShapeDtypeStruct((B, 128), jnp.float32),
        scratch_shapes=[pltpu.VMEM((B, H1), jnp.float32)],
    )


def kernel(snp, snp_ids, node_graph_ids, filters, W1, b1, gamma1, beta1,
           W2, b2, gamma2, beta2, W3, b3):
    f32 = jnp.float32

    table = _make_prep()(snp, filters)

    pad = NODES_PAD - N_NODES
    ids_p = jnp.concatenate(
        [snp_ids, jnp.full((pad,), N_SNPS, jnp.int32)]).reshape(
            TOT_CHUNKS, CHUNK)
    genes_p = jnp.concatenate(
        [node_graph_ids, jnp.full((pad,), N_GENES - 1, jnp.int32)]).reshape(
            TOT_CHUNKS, CHUNK)
    zeros_init = jnp.zeros((N_GENES_PAD, B), f32)

    parts = _make_sc()(table, ids_p, genes_p, zeros_init)
    parts = parts.reshape(2, N_GENES_PAD, B)

    out = _make_mlp()(
        parts, W1,
        b1.reshape(1, H1), gamma1.reshape(1, H1), beta1.reshape(1, H1),
        W2, b2.reshape(1, H2), gamma2.reshape(1, H2), beta2.reshape(1, H2),
        W3.reshape(1, H2), jnp.broadcast_to(b3.reshape(1, 1), (1, 128)),
    )
    preds = out[:, :1]
    return (preds, filters)
